# baseline (speedup 1.0000x reference)
import functools

import jax
import jax.numpy as jnp
from jax import lax
from jax.experimental import pallas as pl
from jax.experimental.pallas import tpu as pltpu
from jax.experimental.pallas import tpu_sc as plsc

_B = 4096
_H = 64
_NC = 2
_NS = 16
_NW = _NC * _NS
_BPW = _B // _NW

_mesh = plsc.VectorSubcoreMesh(core_axis_name="c", subcore_axis_name="s")


@functools.partial(
    pl.kernel,
    mesh=_mesh,
    out_type=jax.ShapeDtypeStruct((_B, 8, _H), jnp.float32),
    scratch_types=[
        pltpu.VMEM((4, _BPW), jnp.int32),
        pltpu.VMEM((_BPW, _H), jnp.float32),
        pltpu.VMEM((_BPW, _H), jnp.float32),
        pltpu.VMEM((_BPW, _H), jnp.float32),
        pltpu.VMEM((_BPW, _H), jnp.float32),
        pltpu.SemaphoreType.DMA,
        pltpu.SemaphoreType.DMA,
    ],
)
def _mdp_embed(s0, s1, s2, s3, i0, i1, i2, i3, table, out,
               idx_v, gb0, gb1, gb2, gb3,
               lsem, ssem):
    wid = lax.axis_index("s") * _NC + lax.axis_index("c")
    base = wid * _BPW
    states = (s0, s1, s2, s3)
    idx_hbm = (i0, i1, i2, i3)
    gbufs = (gb0, gb1, gb2, gb3)

    for i in range(4):
        pltpu.sync_copy(idx_hbm[i].at[pl.ds(base, _BPW)], idx_v.at[i])

    stores = []
    for i in range(4):
        stores.append(pltpu.async_copy(
            states[i].at[pl.ds(base, _BPW)], out.at[pl.ds(base, _BPW), 2 * i], ssem))

    def vec_body(k, _):
        for i in range(4):
            v = idx_v[i, pl.ds(k * 16, 16)]
            for j in range(16):
                row = v[j]
                pltpu.async_copy(table.at[pl.ds(row, 1), :],
                                 gbufs[i].at[pl.ds(k * 16 + j, 1), :], lsem)
        return 0

    lax.fori_loop(0, _BPW // 16, vec_body, 0)

    for i in range(4):
        pltpu.make_async_copy(table.at[pl.ds(0, _BPW), :], gbufs[i], lsem).wait()

    for i in range(4):
        stores.append(pltpu.async_copy(gbufs[i], out.at[pl.ds(base, _BPW), 2 * i + 1], ssem))
    for c in stores:
        c.wait()


def kernel(s0, a0, s1, a1, s2, a2, s3, a3, table):
    i0 = a0.reshape(-1).astype(jnp.int32)
    i1 = a1.reshape(-1).astype(jnp.int32)
    i2 = a2.reshape(-1).astype(jnp.int32)
    i3 = a3.reshape(-1).astype(jnp.int32)
    return _mdp_embed(s0, s1, s2, s3, i0, i1, i2, i3, table)


# speedup vs baseline: 1.0006x; 1.0006x over previous
import functools

import jax
import jax.numpy as jnp
from jax import lax
from jax.experimental import pallas as pl
from jax.experimental.pallas import tpu as pltpu
from jax.experimental.pallas import tpu_sc as plsc

_B = 4096
_H = 64
_NC = 2
_NS = 16
_NW = _NC * _NS
_BPW = _B // _NW
_NSEM = 8
_ROWS_PER_SEM = 4 * _BPW // _NSEM


@functools.partial(
    pl.kernel,
    mesh=plsc.VectorSubcoreMesh(core_axis_name="c", subcore_axis_name="s"),
    out_type=jax.ShapeDtypeStruct((_B, 8, _H), jnp.float32),
    scratch_types=[
        pltpu.VMEM((4, _BPW), jnp.int32),
        pltpu.VMEM((_BPW, _H), jnp.float32),
        pltpu.VMEM((_BPW, _H), jnp.float32),
        pltpu.VMEM((_BPW, _H), jnp.float32),
        pltpu.VMEM((_BPW, _H), jnp.float32),
        [pltpu.SemaphoreType.DMA] * _NSEM,
        pltpu.SemaphoreType.DMA,
    ],
)
def _mdp_embed(s0, s1, s2, s3, i0, i1, i2, i3, table, out,
               idx_v, gb0, gb1, gb2, gb3, gsems, ssem):
    wid = lax.axis_index("s") * _NC + lax.axis_index("c")
    base = wid * _BPW
    states = (s0, s1, s2, s3)
    idx_hbm = (i0, i1, i2, i3)
    gbufs = (gb0, gb1, gb2, gb3)

    for i in range(4):
        pltpu.sync_copy(idx_hbm[i].at[pl.ds(base, _BPW)], idx_v.at[i])

    stores = []
    for i in range(4):
        stores.append(pltpu.async_copy(
            states[i].at[pl.ds(base, _BPW)], out.at[pl.ds(base, _BPW), 2 * i], ssem))

    def vec_body(k, _):
        for i in range(4):
            v = idx_v[i, pl.ds(k * 16, 16)]
            for j in range(16):
                row = v[j]
                pltpu.async_copy(table.at[pl.ds(row, 1), :],
                                 gbufs[i].at[pl.ds(k * 16 + j, 1), :],
                                 gsems[(i * 16 + j) % _NSEM])
        return 0

    lax.fori_loop(0, _BPW // 16, vec_body, 0)

    for k in range(_NSEM):
        pltpu.make_async_copy(table.at[pl.ds(0, _ROWS_PER_SEM), :],
                              gb0.at[pl.ds(0, _ROWS_PER_SEM), :], gsems[k]).wait()

    for i in range(4):
        stores.append(pltpu.async_copy(gbufs[i], out.at[pl.ds(base, _BPW), 2 * i + 1], ssem))
    for c in stores:
        c.wait()


def kernel(s0, a0, s1, a1, s2, a2, s3, a3, table):
    i0 = a0.reshape(-1).astype(jnp.int32)
    i1 = a1.reshape(-1).astype(jnp.int32)
    i2 = a2.reshape(-1).astype(jnp.int32)
    i3 = a3.reshape(-1).astype(jnp.int32)
    return _mdp_embed(s0, s1, s2, s3, i0, i1, i2, i3, table)
